# baseline (speedup 1.0000x reference)
import jax
import jax.numpy as jnp
from jax.experimental import pallas as pl
from jax.experimental.pallas import tpu as pltpu
from jax.experimental.pallas import tpu_sc as plsc

_NUM_CORES = 2
_CHUNK = 256
_SLOTS = 2


def kernel(x, emb_table):
    seq_len = x.shape[1]
    dim = emb_table.shape[1]
    rows_per = seq_len // _NUM_CORES
    n_chunks = rows_per // _CHUNK

    mesh = plsc.ScalarSubcoreMesh(axis_name="c", num_cores=_NUM_CORES)

    @pl.kernel(
        out_type=jax.ShapeDtypeStruct((1, seq_len, dim), emb_table.dtype),
        mesh=mesh,
        scratch_types=[
            pltpu.VMEM_SHARED((_SLOTS, _CHUNK, dim), emb_table.dtype),
            pltpu.SemaphoreType.DMA((_SLOTS,)),
            pltpu.SemaphoreType.DMA((_SLOTS,)),
        ],
    )
    def copy_rows(table_hbm, out_hbm, buf, in_sems, out_sems):
        core = jax.lax.axis_index("c")
        base = core * rows_per

        def in_copy(i, slot):
            return pltpu.make_async_copy(
                table_hbm.at[pl.ds(base + i * _CHUNK, _CHUNK), :],
                buf.at[slot],
                in_sems.at[slot],
            )

        def out_copy(i, slot):
            return pltpu.make_async_copy(
                buf.at[slot],
                out_hbm.at[0].at[pl.ds(base + i * _CHUNK, _CHUNK), :],
                out_sems.at[slot],
            )

        for i in range(min(_SLOTS, n_chunks)):
            in_copy(i, i).start()
        for i in range(n_chunks):
            slot = i % _SLOTS
            in_copy(i, slot).wait()
            out_copy(i, slot).start()
            if i + _SLOTS < n_chunks:
                out_copy(i, slot).wait()
                in_copy(i + _SLOTS, slot).start()
        for i in range(max(n_chunks - _SLOTS, 0), n_chunks):
            out_copy(i, i % _SLOTS).wait()

    return copy_rows(emb_table)


# speedup vs baseline: 1.2919x; 1.2919x over previous
import jax
import jax.numpy as jnp
from jax.experimental import pallas as pl
from jax.experimental.pallas import tpu as pltpu
from jax.experimental.pallas import tpu_sc as plsc

_NUM_SUBCORES = 16
_CHUNK = 128
_SLOTS = 2


def kernel(x, emb_table):
    seq_len = x.shape[1]
    dim = emb_table.shape[1]
    rows_per = seq_len // _NUM_SUBCORES
    n_chunks = rows_per // _CHUNK

    mesh = plsc.VectorSubcoreMesh(core_axis_name="c", subcore_axis_name="s")

    @pl.kernel(
        out_type=jax.ShapeDtypeStruct((1, seq_len, dim), emb_table.dtype),
        mesh=mesh,
        scratch_types=[
            pltpu.VMEM((_SLOTS, _CHUNK, dim), emb_table.dtype),
            pltpu.SemaphoreType.DMA((_SLOTS,)),
            pltpu.SemaphoreType.DMA((_SLOTS,)),
        ],
    )
    def copy_rows(table_hbm, out_hbm, buf, in_sems, out_sems):
        core = jax.lax.axis_index("c")
        sub = jax.lax.axis_index("s")
        base = sub * rows_per

        def in_copy(i, slot):
            return pltpu.make_async_copy(
                table_hbm.at[pl.ds(base + i * _CHUNK, _CHUNK), :],
                buf.at[slot],
                in_sems.at[slot],
            )

        def out_copy(i, slot):
            return pltpu.make_async_copy(
                buf.at[slot],
                out_hbm.at[0].at[pl.ds(base + i * _CHUNK, _CHUNK), :],
                out_sems.at[slot],
            )

        @pl.when(core == 0)
        def _():
            for i in range(min(_SLOTS, n_chunks)):
                in_copy(i, i).start()
            for i in range(n_chunks):
                slot = i % _SLOTS
                in_copy(i, slot).wait()
                out_copy(i, slot).start()
                if i + _SLOTS < n_chunks:
                    out_copy(i, slot).wait()
                    in_copy(i + _SLOTS, slot).start()
            for i in range(max(n_chunks - _SLOTS, 0), n_chunks):
                out_copy(i, i % _SLOTS).wait()

    return copy_rows(emb_table)


# speedup vs baseline: 1.4228x; 1.1013x over previous
import jax
import jax.numpy as jnp
from jax.experimental import pallas as pl
from jax.experimental.pallas import tpu as pltpu
from jax.experimental.pallas import tpu_sc as plsc

_NUM_CORES = 2
_NUM_SUBCORES = 16
_CHUNK = 128
_SLOTS = 2


def kernel(x, emb_table):
    seq_len = x.shape[1]
    dim = emb_table.shape[1]
    num_workers = _NUM_CORES * _NUM_SUBCORES
    rows_per = seq_len // num_workers
    assert rows_per * num_workers == seq_len and rows_per % _CHUNK == 0
    n_chunks = rows_per // _CHUNK

    mesh = plsc.VectorSubcoreMesh(core_axis_name="c", subcore_axis_name="s")

    @pl.kernel(
        out_type=jax.ShapeDtypeStruct((1, seq_len, dim), emb_table.dtype),
        mesh=mesh,
        scratch_types=[
            pltpu.VMEM((_SLOTS, _CHUNK, dim), emb_table.dtype),
            pltpu.SemaphoreType.DMA((_SLOTS,)),
            pltpu.SemaphoreType.DMA((_SLOTS,)),
        ],
    )
    def copy_rows(table_hbm, out_hbm, buf, in_sems, out_sems):
        core = jax.lax.axis_index("c")
        sub = jax.lax.axis_index("s")
        base = (core * _NUM_SUBCORES + sub) * rows_per

        def in_copy(i, slot):
            return pltpu.make_async_copy(
                table_hbm.at[pl.ds(base + i * _CHUNK, _CHUNK), :],
                buf.at[slot],
                in_sems.at[slot],
            )

        def out_copy(i, slot):
            return pltpu.make_async_copy(
                buf.at[slot],
                out_hbm.at[0].at[pl.ds(base + i * _CHUNK, _CHUNK), :],
                out_sems.at[slot],
            )

        for i in range(min(_SLOTS, n_chunks)):
            in_copy(i, i).start()
        for i in range(n_chunks):
            slot = i % _SLOTS
            in_copy(i, slot).wait()
            out_copy(i, slot).start()
            if i + _SLOTS < n_chunks:
                out_copy(i, slot).wait()
                in_copy(i + _SLOTS, slot).start()
        for i in range(max(n_chunks - _SLOTS, 0), n_chunks):
            out_copy(i, i % _SLOTS).wait()

    return copy_rows(emb_table)
